# baseline (speedup 1.0000x reference)
import functools
import jax
import jax.numpy as jnp
from jax import lax
from jax.experimental import pallas as pl
from jax.experimental.pallas import tpu as pltpu
from jax.experimental.pallas import tpu_sc as plsc

_P, _C, _D = 2048, 128, 128
_NC, _NS = 2, 16
_NW = _NC * _NS
_CHUNK = 128
_CHW = _C // _NW


def _sc_body(time_hbm, chan_hbm, out_hbm, time_v, chan_v, out_v):
    wid = lax.axis_index("s") * _NC + lax.axis_index("c")
    c0 = wid * _CHW
    pltpu.sync_copy(chan_hbm.at[pl.ds(c0, _CHW)], chan_v)

    def chunk_body(k, carry):
        pltpu.sync_copy(time_hbm.at[pl.ds(k * _CHUNK, _CHUNK)], time_v)
        for ch in range(_CHW):
            chan_vecs = [chan_v[ch, pl.ds(d * 16, 16)] for d in range(_D // 16)]

            def p_body(p, c2):
                for d in range(_D // 16):
                    out_v[p, pl.ds(d * 16, 16)] = (
                        time_v[p, pl.ds(d * 16, 16)] + chan_vecs[d]
                    )
                return c2

            lax.fori_loop(0, _CHUNK, p_body, 0)
            pltpu.sync_copy(
                out_v, out_hbm.at[pl.ds((c0 + ch) * _P + k * _CHUNK, _CHUNK)]
            )
        return carry

    lax.fori_loop(0, _P // _CHUNK, chunk_body, 0)


def kernel(num_patches_per_channel, num_channels, time_embed, channel_embed):
    del num_patches_per_channel, num_channels
    mesh = plsc.VectorSubcoreMesh(core_axis_name="c", subcore_axis_name="s")
    run = functools.partial(
        pl.kernel,
        out_type=jax.ShapeDtypeStruct((_C * _P, _D), jnp.float32),
        mesh=mesh,
        scratch_types=[
            pltpu.VMEM((_CHUNK, _D), jnp.float32),
            pltpu.VMEM((_CHW, _D), jnp.float32),
            pltpu.VMEM((_CHUNK, _D), jnp.float32),
        ],
    )(_sc_body)
    return run(time_embed, channel_embed)


# speedup vs baseline: 1.1715x; 1.1715x over previous
import functools
import jax
import jax.numpy as jnp
from jax import lax
from jax.experimental import pallas as pl
from jax.experimental.pallas import tpu as pltpu
from jax.experimental.pallas import tpu_sc as plsc

_P, _C, _D = 2048, 128, 128
_NC, _NS = 2, 16
_NW = _NC * _NS
_CHUNK = 128
_NCHUNK = _P // _CHUNK
_CHW = _C // _NW
_DG = _D // 16


def _sc_body(time_hbm, chan_hbm, out_hbm, time_v, chan_v, out_v,
             sem_in0, sem_in1, sem_out0, sem_out1, sem_out2, sem_out3):
    sems_in = (sem_in0, sem_in1)
    sems_out = (sem_out0, sem_out1, sem_out2, sem_out3)
    wid = lax.axis_index("s") * _NC + lax.axis_index("c")
    c0 = wid * _CHW
    pltpu.sync_copy(chan_hbm.at[pl.ds(c0, _CHW)], chan_v)

    def start_load(k, b):
        pltpu.async_copy(
            time_hbm.at[pl.ds(k * _CHUNK, _CHUNK)], time_v.at[b], sems_in[b])

    def wait_load(k, b):
        pltpu.make_async_copy(
            time_hbm.at[pl.ds(k * _CHUNK, _CHUNK)], time_v.at[b],
            sems_in[b]).wait()

    def out_slice(k, ch):
        return out_hbm.at[pl.ds((c0 + ch) * _P + k * _CHUNK, _CHUNK)]

    def fire_store(k, ch):
        pltpu.async_copy(out_v.at[ch], out_slice(k, ch), sems_out[ch])

    def wait_store(k, ch):
        pltpu.make_async_copy(out_v.at[ch], out_slice(k, ch),
                              sems_out[ch]).wait()

    def compute(b, ch):
        cvecs = [chan_v[ch, pl.ds(d * 16, 16)] for d in range(_DG)]

        @pl.loop(0, _CHUNK, unroll=4)
        def p_body(p):
            for d in range(_DG):
                out_v[ch, p, pl.ds(d * 16, 16)] = (
                    time_v[b, p, pl.ds(d * 16, 16)] + cvecs[d])

    start_load(0, 0)
    wait_load(0, 0)
    start_load(1, 1)
    for ch in range(_CHW):
        compute(0, ch)
        fire_store(0, ch)

    def pair(i, carry):
        k1 = 2 * i + 1
        wait_load(k1, 1)
        start_load(k1 + 1, 0)
        for ch in range(_CHW):
            wait_store(k1, ch)
            compute(1, ch)
            fire_store(k1, ch)
        k2 = 2 * i + 2
        wait_load(k2, 0)
        start_load(k2 + 1, 1)
        for ch in range(_CHW):
            wait_store(k2, ch)
            compute(0, ch)
            fire_store(k2, ch)
        return carry

    lax.fori_loop(0, (_NCHUNK - 2) // 2, pair, 0)

    k_last = _NCHUNK - 1
    wait_load(k_last, 1)
    for ch in range(_CHW):
        wait_store(k_last - 1, ch)
        compute(1, ch)
        fire_store(k_last, ch)
    for ch in range(_CHW):
        wait_store(k_last, ch)


def kernel(num_patches_per_channel, num_channels, time_embed, channel_embed):
    del num_patches_per_channel, num_channels
    mesh = plsc.VectorSubcoreMesh(core_axis_name="c", subcore_axis_name="s")
    run = functools.partial(
        pl.kernel,
        out_type=jax.ShapeDtypeStruct((_C * _P, _D), jnp.float32),
        mesh=mesh,
        scratch_types=[
            pltpu.VMEM((2, _CHUNK, _D), jnp.float32),
            pltpu.VMEM((_CHW, _D), jnp.float32),
            pltpu.VMEM((_CHW, _CHUNK, _D), jnp.float32),
            pltpu.SemaphoreType.DMA,
            pltpu.SemaphoreType.DMA,
            pltpu.SemaphoreType.DMA,
            pltpu.SemaphoreType.DMA,
            pltpu.SemaphoreType.DMA,
            pltpu.SemaphoreType.DMA,
        ],
    )(_sc_body)
    return run(time_embed, channel_embed)
